# baseline (speedup 1.0000x reference)
import functools

import jax
import jax.numpy as jnp
from jax import lax
from jax.experimental import pallas as pl
from jax.experimental.pallas import tpu as pltpu
from jax.experimental.pallas import tpu_sc as plsc

N = 4_000_000
CHUNK = 3_200
NUM_CHUNKS = N // CHUNK
L = 16
NC, NS = 2, 16
NW = NC * NS
N_SP = 119
NK_MAX = -(-NUM_CHUNKS // NW)
NK_EVEN = NK_MAX + (NK_MAX % 2)


@functools.partial(
    pl.kernel,
    out_type=jax.ShapeDtypeStruct((1, N), jnp.float32),
    mesh=plsc.VectorSubcoreMesh(core_axis_name="c", subcore_axis_name="s"),
    scratch_types=[
        pltpu.VMEM((1, CHUNK), jnp.float32),
        pltpu.VMEM((1, CHUNK), jnp.float32),
        pltpu.VMEM((CHUNK,), jnp.int32),
        pltpu.VMEM((CHUNK,), jnp.int32),
        pltpu.VMEM((1, CHUNK), jnp.float32),
        pltpu.VMEM((1, CHUNK), jnp.float32),
        pltpu.VMEM((1, N_SP), jnp.float32),
        pltpu.VMEM((1, N_SP), jnp.float32),
        pltpu.SemaphoreType.DMA,
        pltpu.SemaphoreType.DMA,
        pltpu.SemaphoreType.DMA,
        pltpu.SemaphoreType.DMA,
        pltpu.SemaphoreType.DMA,
        pltpu.SemaphoreType.DMA,
    ],
    compiler_params=pltpu.CompilerParams(
        needs_layout_passes=False, use_tc_tiling_on_sc=True),
)
def _scale_shift_sc(x_hbm, z_hbm, s_hbm, b_hbm, out_hbm,
                    xv0, xv1, zv0, zv1, ov0, ov1, ts, tb,
                    sx0, sx1, sz0, sz1, so0, so1):
    xv, zv, ov = (xv0, xv1), (zv0, zv1), (ov0, ov1)
    sx, sz, so = (sx0, sx1), (sz0, sz1), (so0, so1)
    wid = lax.axis_index("s") * NC + lax.axis_index("c")
    nk = (NUM_CHUNKS - wid + NW - 1) // NW
    pltpu.sync_copy(s_hbm, ts)
    pltpu.sync_copy(b_hbm, tb)
    zero = jnp.zeros((L,), jnp.int32)

    def start_in(k, b):
        off = (wid + k * NW) * CHUNK
        pltpu.async_copy(x_hbm.at[:, pl.ds(off, CHUNK)], xv[b], sx[b])
        pltpu.async_copy(z_hbm.at[pl.ds(off, CHUNK)], zv[b], sz[b])

    def wait_in(b):
        pltpu.make_async_copy(x_hbm.at[:, pl.ds(0, CHUNK)], xv[b], sx[b]).wait()
        pltpu.make_async_copy(z_hbm.at[pl.ds(0, CHUNK)], zv[b], sz[b]).wait()

    def start_out(k, b):
        off = (wid + k * NW) * CHUNK
        pltpu.async_copy(ov[b], out_hbm.at[:, pl.ds(off, CHUNK)], so[b])

    def wait_out(b):
        pltpu.make_async_copy(ov[b], out_hbm.at[:, pl.ds(0, CHUNK)], so[b]).wait()

    start_in(0, 0)

    @pl.when(nk > 1)
    def _():
        start_in(1, 1)

    @pl.loop(0, NK_EVEN, step=2)
    def _pair(kk):
        for b in (0, 1):
            k = kk + b

            @pl.when(k < nk)
            def _():
                wait_in(b)

                @pl.when(k >= 2)
                def _():
                    wait_out(b)

                @plsc.parallel_loop(0, CHUNK // L, unroll=16)
                def _vec(i):
                    sl = pl.ds(i * L, L)
                    idx = zv[b][sl]
                    s = plsc.load_gather(ts, [zero, idx])
                    sh = plsc.load_gather(tb, [zero, idx])
                    ov[b][0, sl] = s * xv[b][0, sl] + sh

                start_out(k, b)

                @pl.when(k + 2 < nk)
                def _():
                    start_in(k + 2, b)

    for b in (0, 1):
        @pl.when(nk > b)
        def _():
            wait_out(b)


def kernel(x, Z, scale_param, shift_param):
    out = _scale_shift_sc(x.T, Z.astype(jnp.int32), scale_param.T,
                          shift_param.T)
    return out.T
